# baseline (speedup 1.0000x reference)
import functools

import jax
import jax.numpy as jnp
from jax.experimental import pallas as pl
from jax.experimental.pallas import tpu as pltpu

K = 10
N = 2048
ROW_TILE = 2048


def _chain_mins(d, k):
    m = jnp.min(d, axis=1, keepdims=True)
    cols = [m]
    for _ in range(k - 1):
        m = jnp.min(jnp.where(d > m, d, jnp.inf), axis=1, keepdims=True)
        cols.append(m)
    return cols


def _topk_cands(d, k):
    if d.shape[1] <= 256 or k <= 2:
        return _chain_mins(d, k)
    h = d.shape[1] // 2
    a, b = d[:, :h], d[:, h:]
    lo = jnp.minimum(a, b)
    hi = jnp.maximum(a, b)
    return _topk_cands(lo, k) + _topk_cands(hi, (k - 1) // 2 + 1)


def _pointnet_kernel(x_nc_ref, x_cn_ref, w1_ref, b1_ref, w2_ref, b2_ref,
                     w3_ref, b3_ref, w4a_ref, w4b_ref, b4_ref, w5_ref,
                     b5_ref, w6_ref, b6_ref, out_ref, feat_ref):
    n = x_nc_ref.shape[1]
    xcn = x_cn_ref[0, :, :]
    ncol = jnp.sum(xcn * xcn, axis=0, keepdims=True)
    xc5 = jnp.concatenate([-2.0 * xcn, jnp.ones((1, n), jnp.float32), ncol],
                          axis=0)
    for r0 in range(0, n, ROW_TILE):
        xr3 = x_nc_ref[0, pl.ds(r0, ROW_TILE), :]
        nrow = jnp.sum(xr3 * xr3, axis=1, keepdims=True)
        xr5 = jnp.concatenate(
            [xr3, nrow, jnp.ones((ROW_TILE, 1), jnp.float32)], axis=1)
        d = jnp.dot(xr5, xc5, preferred_element_type=jnp.float32)
        c = jnp.concatenate(_topk_cands(d, K), axis=1)
        ks = _chain_mins(c, K)
        xr3 = x_nc_ref[0, pl.ds(r0, ROW_TILE), :]
        feat_ref[pl.ds(r0, ROW_TILE), :] = jnp.concatenate([xr3] + ks, axis=1)

    f = feat_ref[:, :]
    h = jnp.maximum(jnp.dot(f, w1_ref[:, :],
                            preferred_element_type=jnp.float32) + b1_ref[:, :], 0.0)
    x1 = jnp.maximum(jnp.dot(h, w2_ref[:, :],
                             preferred_element_type=jnp.float32) + b2_ref[:, :], 0.0)
    x2 = jnp.maximum(jnp.dot(x1, w3_ref[:, :],
                             preferred_element_type=jnp.float32) + b3_ref[:, :], 0.0)
    pool = jnp.mean(x2, axis=0, keepdims=True)
    o = jnp.dot(x1, w4a_ref[:, :], preferred_element_type=jnp.float32)
    o = o + jnp.dot(pool, w4b_ref[:, :], preferred_element_type=jnp.float32)
    o = jnp.maximum(o + b4_ref[:, :], 0.0)
    o = jnp.maximum(jnp.dot(o, w5_ref[:, :],
                            preferred_element_type=jnp.float32) + b5_ref[:, :], 0.0)
    logits = jnp.dot(o, w6_ref[:, :],
                     preferred_element_type=jnp.float32) + b6_ref[:, :]
    mx = jnp.max(logits, axis=1, keepdims=True)
    lse = mx + jnp.log(jnp.sum(jnp.exp(logits - mx), axis=1, keepdims=True))
    out_ref[0, :, :] = logits - lse


@jax.jit
def kernel(x, W1, b1, W2, b2, W3, b3, W4, b4, W5, b5, W6, b6):
    B, n, C = x.shape
    GF = W2.shape[0]
    x_cn = jnp.transpose(x, (0, 2, 1))
    w1t = jnp.transpose(W1)
    w2t = jnp.transpose(W2)
    w3t = jnp.transpose(W3)
    w4at = jnp.transpose(W4[:, :GF])
    w4bt = jnp.transpose(W4[:, GF:])
    w5t = jnp.transpose(W5)
    w6t = jnp.transpose(W6)
    biases = [b.reshape(1, -1) for b in (b1, b2, b3, b4, b5, b6)]

    full = lambda a: pl.BlockSpec(a.shape, lambda b: (0,) * a.ndim)
    in_specs = [
            pl.BlockSpec((1, n, C), lambda b: (b, 0, 0)),
            pl.BlockSpec((1, C, n), lambda b: (b, 0, 0)),
            full(w1t), full(biases[0]), full(w2t), full(biases[1]),
            full(w3t), full(biases[2]), full(w4at), full(w4bt),
            full(biases[3]), full(w5t), full(biases[4]), full(w6t),
            full(biases[5]),
    ]
    out = pl.pallas_call(
        _pointnet_kernel,
        grid=(B,),
        in_specs=in_specs,
        out_specs=pl.BlockSpec((1, n, 2), lambda b: (b, 0, 0)),
        out_shape=jax.ShapeDtypeStruct((B, n, 2), jnp.float32),
        scratch_shapes=[pltpu.VMEM((n, 3 + K), jnp.float32)],
        compiler_params=pltpu.CompilerParams(
            dimension_semantics=("parallel",),
        ),
    )(x, x_cn, w1t, biases[0], w2t, biases[1], w3t, biases[2],
      w4at, w4bt, biases[3], w5t, biases[4], w6t, biases[5])
    return out


# speedup vs baseline: 1.0732x; 1.0732x over previous
import jax
import jax.numpy as jnp
from jax import lax
from jax.experimental import pallas as pl
from jax.experimental.pallas import tpu as pltpu

K = 10
N = 2048
ROW_TILE = 1024
LEAF = 512


def _chain_mins(d, k):
    m = jnp.min(d, axis=-1, keepdims=True)
    cols = [m]
    for _ in range(k - 1):
        m = jnp.min(jnp.where(d > m, d, jnp.inf), axis=-1, keepdims=True)
        cols.append(m)
    return cols


def _topk_cands(d, k):
    if d.shape[-1] <= LEAF or k <= 2:
        return _chain_mins(d, k)
    h = d.shape[-1] // 2
    a, b = d[..., :h], d[..., h:]
    lo = jnp.minimum(a, b)
    hi = jnp.maximum(a, b)
    return _topk_cands(lo, k) + _topk_cands(hi, (k - 1) // 2 + 1)


def _pointnet_kernel(x_nc_ref, x_cn_ref, w1_ref, b1_ref, w2_ref, b2_ref,
                     w3_ref, b3_ref, w4a_ref, w4b_ref, b4_ref, w5_ref,
                     b5_ref, w6_ref, b6_ref, out_ref, feat_ref):
    nb = x_nc_ref.shape[0]
    n = x_nc_ref.shape[1]
    xcn = x_cn_ref[:, :, :]
    ncol = jnp.sum(xcn * xcn, axis=1, keepdims=True)
    xc5 = jnp.concatenate(
        [-2.0 * xcn, jnp.ones((nb, 1, n), jnp.float32), ncol], axis=1)
    for r0 in range(0, n, ROW_TILE):
        xr3 = x_nc_ref[:, pl.ds(r0, ROW_TILE), :]
        nrow = jnp.sum(xr3 * xr3, axis=-1, keepdims=True)
        xr5 = jnp.concatenate(
            [xr3, nrow, jnp.ones((nb, ROW_TILE, 1), jnp.float32)], axis=-1)
        d = lax.dot_general(xr5, xc5, (((2,), (1,)), ((0,), (0,))),
                            preferred_element_type=jnp.float32)
        c = jnp.concatenate(_topk_cands(d, K), axis=-1)
        ks = _chain_mins(c, K)
        feat_ref[:, pl.ds(r0, ROW_TILE), :] = jnp.concatenate(
            [xr3] + ks, axis=-1)

    for bb in range(nb):
        f = feat_ref[bb, :, :]
        h = jnp.maximum(jnp.dot(f, w1_ref[:, :],
                                preferred_element_type=jnp.float32) + b1_ref[:, :], 0.0)
        x1 = jnp.maximum(jnp.dot(h, w2_ref[:, :],
                                 preferred_element_type=jnp.float32) + b2_ref[:, :], 0.0)
        x2 = jnp.maximum(jnp.dot(x1, w3_ref[:, :],
                                 preferred_element_type=jnp.float32) + b3_ref[:, :], 0.0)
        pool = jnp.mean(x2, axis=0, keepdims=True)
        o = jnp.dot(x1, w4a_ref[:, :], preferred_element_type=jnp.float32)
        o = o + jnp.dot(pool, w4b_ref[:, :], preferred_element_type=jnp.float32)
        o = jnp.maximum(o + b4_ref[:, :], 0.0)
        o = jnp.maximum(jnp.dot(o, w5_ref[:, :],
                                preferred_element_type=jnp.float32) + b5_ref[:, :], 0.0)
        logits = jnp.dot(o, w6_ref[:, :],
                         preferred_element_type=jnp.float32) + b6_ref[:, :]
        mx = jnp.max(logits, axis=1, keepdims=True)
        lse = mx + jnp.log(jnp.sum(jnp.exp(logits - mx), axis=1, keepdims=True))
        out_ref[bb, :, :] = logits - lse


@jax.jit
def kernel(x, W1, b1, W2, b2, W3, b3, W4, b4, W5, b5, W6, b6):
    B, n, C = x.shape
    GF = W2.shape[0]
    PB = 2
    x_cn = jnp.transpose(x, (0, 2, 1))
    w1t = jnp.transpose(W1)
    w2t = jnp.transpose(W2)
    w3t = jnp.transpose(W3)
    w4at = jnp.transpose(W4[:, :GF])
    w4bt = jnp.transpose(W4[:, GF:])
    w5t = jnp.transpose(W5)
    w6t = jnp.transpose(W6)
    biases = [b.reshape(1, -1) for b in (b1, b2, b3, b4, b5, b6)]

    full = lambda a: pl.BlockSpec(a.shape, lambda b: (0,) * a.ndim)
    in_specs = [
            pl.BlockSpec((PB, n, C), lambda b: (b, 0, 0)),
            pl.BlockSpec((PB, C, n), lambda b: (b, 0, 0)),
            full(w1t), full(biases[0]), full(w2t), full(biases[1]),
            full(w3t), full(biases[2]), full(w4at), full(w4bt),
            full(biases[3]), full(w5t), full(biases[4]), full(w6t),
            full(biases[5]),
    ]
    out = pl.pallas_call(
        _pointnet_kernel,
        grid=(B // PB,),
        in_specs=in_specs,
        out_specs=pl.BlockSpec((PB, n, 2), lambda b: (b, 0, 0)),
        out_shape=jax.ShapeDtypeStruct((B, n, 2), jnp.float32),
        scratch_shapes=[pltpu.VMEM((PB, n, 3 + K), jnp.float32)],
        compiler_params=pltpu.CompilerParams(
            dimension_semantics=("parallel",),
        ),
    )(x, x_cn, w1t, biases[0], w2t, biases[1], w3t, biases[2],
      w4at, w4bt, biases[3], w5t, biases[4], w6t, biases[5])
    return out
